# baseline (speedup 1.0000x reference)
import jax
import jax.numpy as jnp
from jax.experimental import pallas as pl
from jax.experimental.pallas import tpu as pltpu

N = 1024
F = 64
B = 16
BB = 8
M = 5
W_BB = BB * F


def _dotT(a, b):
    return jax.lax.dot_general(
        a, b, (((0,), (0,)), ((), ())), preferred_element_type=jnp.float32)


def _dot(a, b):
    return jax.lax.dot_general(
        a, b, (((1,), (0,)), ((), ())), preferred_element_type=jnp.float32)


def _cell_kernel(xcat_ref, adj_ref, wg_ref, bg_ref, wc_ref, bc_ref,
                 out_ref, s1_ref, s2_ref):
    step = pl.program_id(0)
    bf = jnp.bfloat16

    @pl.when(step == 0)
    def _build_supports():
        a = adj_ref[...]
        ones = jnp.ones((N, 1), dtype=bf)
        d = _dot(a, ones)
        dinv = jnp.where(d > 0.0, 1.0 / d, 0.0).astype(bf)
        s1_ref[...] = dinv * a
        d2 = _dotT(a, ones).reshape(1, N)
        d2inv = jnp.where(d2 > 0.0, 1.0 / d2, 0.0).astype(bf)
        s2_ref[...] = a * d2inv

    rw = s1_ref[...]
    s2 = s2_ref[...]

    x0 = jnp.concatenate(
        [xcat_ref[b].astype(bf) for b in range(BB)], axis=1)

    y1 = _dotT(rw, x0).astype(bf)
    y2 = (2.0 * _dotT(rw, y1) - x0).astype(bf)
    z1 = _dot(s2, x0).astype(bf)
    z2 = (2.0 * _dot(s2, z1) - x0).astype(bf)

    wg = wg_ref[...]
    bg = bg_ref[...]
    wc = wc_ref[...]
    bc = bc_ref[...]

    st2p_parts = []
    u_parts = []
    for b in range(BB):
        lo, hi = b * 2 * F, (b + 1) * 2 * F
        xb = jnp.concatenate(
            [x0[:, lo:hi], y1[:, lo:hi], y2[:, lo:hi], z1[:, lo:hi],
             z2[:, lo:hi]], axis=1)
        val = jax.nn.sigmoid(_dot(xb, wg) + bg)
        u_parts.append(val[:, F:].astype(bf))
        st2p_parts.append((val[:, :F] * xcat_ref[b][:, F:]).astype(bf))

    st2pb = jnp.concatenate(st2p_parts, axis=1)
    r1 = _dotT(rw, st2pb).astype(bf)
    r2 = (2.0 * _dotT(rw, r1) - st2pb).astype(bf)
    r3 = _dot(s2, st2pb).astype(bf)
    r4 = (2.0 * _dot(s2, r3) - st2pb).astype(bf)

    for b in range(BB):
        lo = b * 2 * F
        slo, shi = b * F, (b + 1) * F
        xb = jnp.concatenate(
            [x0[:, lo:lo + F], st2pb[:, slo:shi],
             y1[:, lo:lo + F], r1[:, slo:shi],
             y2[:, lo:lo + F], r2[:, slo:shi],
             z1[:, lo:lo + F], r3[:, slo:shi],
             z2[:, lo:lo + F], r4[:, slo:shi]], axis=1)
        c = jnp.tanh(_dot(xb, wc) + bc)
        u = u_parts[b].astype(jnp.float32)
        out_ref[b] = u * xcat_ref[b][:, F:] + (1.0 - u) * c


def kernel(inputs, hx, adj, W_gate, b_gate, W_c, b_c):
    xcat = jnp.concatenate(
        [inputs.reshape(B, N, F), hx.reshape(B, N, F)], axis=2)
    wg = W_gate.reshape(2 * F, M, 2 * F).transpose(1, 0, 2).reshape(
        M * 2 * F, 2 * F).astype(jnp.bfloat16)
    wc = W_c.reshape(2 * F, M, F).transpose(1, 0, 2).reshape(
        M * 2 * F, F).astype(jnp.bfloat16)
    bg = b_gate.reshape(1, 2 * F)
    bc = b_c.reshape(1, F)

    out = pl.pallas_call(
        _cell_kernel,
        grid=(B // BB,),
        in_specs=[
            pl.BlockSpec((BB, N, 2 * F), lambda i: (i, 0, 0)),
            pl.BlockSpec((N, N), lambda i: (0, 0)),
            pl.BlockSpec((M * 2 * F, 2 * F), lambda i: (0, 0)),
            pl.BlockSpec((1, 2 * F), lambda i: (0, 0)),
            pl.BlockSpec((M * 2 * F, F), lambda i: (0, 0)),
            pl.BlockSpec((1, F), lambda i: (0, 0)),
        ],
        out_specs=pl.BlockSpec((BB, N, F), lambda i: (i, 0, 0)),
        out_shape=jax.ShapeDtypeStruct((B, N, F), jnp.float32),
        scratch_shapes=[
            pltpu.VMEM((N, N), jnp.bfloat16),
            pltpu.VMEM((N, N), jnp.bfloat16),
        ],
    )(xcat, adj.astype(jnp.bfloat16), wg, bg, wc, bc)
    return out.reshape(B, N * F)


# speedup vs baseline: 1.0692x; 1.0692x over previous
import jax
import jax.numpy as jnp
from jax.experimental import pallas as pl
from jax.experimental.pallas import tpu as pltpu

N = 1024
F = 64
B = 16
BB = 8
M = 5
W_BB = BB * F


def _dotT(a, b):
    return jax.lax.dot_general(
        a, b, (((0,), (0,)), ((), ())), preferred_element_type=jnp.float32)


def _dot(a, b):
    return jax.lax.dot_general(
        a, b, (((1,), (0,)), ((), ())), preferred_element_type=jnp.float32)


def _supports_kernel(adj_ref, s1_ref, s2_ref):
    a = adj_ref[...]
    d = jnp.sum(a, axis=1, keepdims=True)
    dinv = jnp.where(d > 0.0, 1.0 / d, 0.0)
    s1_ref[...] = (dinv * a).astype(jnp.bfloat16)
    d2 = jnp.sum(a, axis=0, keepdims=True)
    d2inv = jnp.where(d2 > 0.0, 1.0 / d2, 0.0)
    s2_ref[...] = (a * d2inv).astype(jnp.bfloat16)


def _cell_kernel(inp_ref, hx_ref, s1_ref, s2_ref, wg_ref, bg_ref, wc_ref,
                 bc_ref, out_ref):
    bf = jnp.bfloat16
    rw = s1_ref[...]
    s2 = s2_ref[...]

    x0 = jnp.concatenate(
        [inp_ref[b].astype(bf) for b in range(BB)]
        + [hx_ref[b].astype(bf) for b in range(BB)], axis=1)
    inp2b = x0[:, :W_BB]
    st2b = x0[:, W_BB:]

    y1 = _dotT(rw, x0).astype(bf)
    y2 = (2.0 * _dotT(rw, y1) - x0).astype(bf)
    z1 = _dot(s2, x0).astype(bf)
    z2 = (2.0 * _dot(s2, z1) - x0).astype(bf)

    wg = wg_ref[...]
    bg = bg_ref[...]
    wc = wc_ref[...]
    bc = bc_ref[...]

    st2p_parts = []
    u_parts = []
    for b in range(BB):
        lo, hi = b * F, (b + 1) * F
        slo, shi = W_BB + lo, W_BB + hi
        xb = jnp.concatenate(
            [inp2b[:, lo:hi], st2b[:, lo:hi], y1[:, lo:hi], y1[:, slo:shi],
             y2[:, lo:hi], y2[:, slo:shi], z1[:, lo:hi], z1[:, slo:shi],
             z2[:, lo:hi], z2[:, slo:shi]], axis=1)
        val = jax.nn.sigmoid(_dot(xb, wg) + bg)
        u_parts.append(val[:, F:].astype(bf))
        st2p_parts.append((val[:, :F] * hx_ref[b]).astype(bf))

    st2pb = jnp.concatenate(st2p_parts, axis=1)
    r1 = _dotT(rw, st2pb).astype(bf)
    r2 = (2.0 * _dotT(rw, r1) - st2pb).astype(bf)
    r3 = _dot(s2, st2pb).astype(bf)
    r4 = (2.0 * _dot(s2, r3) - st2pb).astype(bf)

    for b in range(BB):
        lo, hi = b * F, (b + 1) * F
        xb = jnp.concatenate(
            [inp2b[:, lo:hi], st2pb[:, lo:hi], y1[:, lo:hi], r1[:, lo:hi],
             y2[:, lo:hi], r2[:, lo:hi], z1[:, lo:hi], r3[:, lo:hi],
             z2[:, lo:hi], r4[:, lo:hi]], axis=1)
        c = jnp.tanh(_dot(xb, wc) + bc)
        u = u_parts[b].astype(jnp.float32)
        out_ref[b] = u * hx_ref[b] + (1.0 - u) * c


def kernel(inputs, hx, adj, W_gate, b_gate, W_c, b_c):
    inp3 = inputs.reshape(B, N, F)
    hx3 = hx.reshape(B, N, F)
    wg = W_gate.reshape(2 * F, M, 2 * F).transpose(1, 0, 2).reshape(
        M * 2 * F, 2 * F).astype(jnp.bfloat16)
    wc = W_c.reshape(2 * F, M, F).transpose(1, 0, 2).reshape(
        M * 2 * F, F).astype(jnp.bfloat16)
    bg = b_gate.reshape(1, 2 * F)
    bc = b_c.reshape(1, F)

    s1, s2 = pl.pallas_call(
        _supports_kernel,
        out_shape=[
            jax.ShapeDtypeStruct((N, N), jnp.bfloat16),
            jax.ShapeDtypeStruct((N, N), jnp.bfloat16),
        ],
    )(adj)

    out = pl.pallas_call(
        _cell_kernel,
        grid=(B // BB,),
        in_specs=[
            pl.BlockSpec((BB, N, F), lambda i: (i, 0, 0)),
            pl.BlockSpec((BB, N, F), lambda i: (i, 0, 0)),
            pl.BlockSpec((N, N), lambda i: (0, 0)),
            pl.BlockSpec((N, N), lambda i: (0, 0)),
            pl.BlockSpec((M * 2 * F, 2 * F), lambda i: (0, 0)),
            pl.BlockSpec((1, 2 * F), lambda i: (0, 0)),
            pl.BlockSpec((M * 2 * F, F), lambda i: (0, 0)),
            pl.BlockSpec((1, F), lambda i: (0, 0)),
        ],
        out_specs=pl.BlockSpec((BB, N, F), lambda i: (i, 0, 0)),
        out_shape=jax.ShapeDtypeStruct((B, N, F), jnp.float32),
    )(inp3, hx3, s1, s2, wg, bg, wc, bc)
    return out.reshape(B, N * F)
